# baseline (speedup 1.0000x reference)
import functools

import numpy as np
import jax
import jax.numpy as jnp
from jax import lax
from jax.experimental import pallas as pl
from jax.experimental.pallas import tpu as pltpu
from jax.experimental.pallas import tpu_sc as plsc

_N_FREQ = 10
_SINE_DIM = _N_FREQ * 2
_D_EMB = 20
_BLK_W = 16

_FREQS = np.array(
    [[1.0 / 1000.0 ** (2.0 * i / _N_FREQ)] * 2 for i in range(1, _N_FREQ + 1)],
    dtype=np.float32,
).reshape(1, -1)
_SHIFTS = np.array(
    [[0.0, np.pi / 2.0] for _ in range(1, _N_FREQ + 1)], dtype=np.float32
).reshape(1, -1)

_NC = 2
_NS = 16
_NW = _NC * _NS
_CHUNK = 128


def _sc_gather(table, idx):
    N = idx.shape[0]
    D = table.shape[1]
    bpw = N // _NW
    nch = bpw // _CHUNK
    idx3 = idx.reshape(_NW, nch, _CHUNK)
    mesh = plsc.VectorSubcoreMesh(core_axis_name="c", subcore_axis_name="s")

    @functools.partial(
        pl.kernel,
        mesh=mesh,
        out_type=jax.ShapeDtypeStruct((N, D), jnp.float32),
        scratch_types=[
            pltpu.VMEM((nch, _CHUNK), jnp.int32),
            pltpu.VMEM((bpw, D), jnp.float32),
            pltpu.SemaphoreType.DMA,
        ],
        compiler_params=pltpu.CompilerParams(use_tc_tiling_on_sc=False),
    )
    def gather_kernel(idx_hbm, table_hbm, out_hbm, idx_v, rows_v, sem):
        wid = lax.axis_index("s") * _NC + lax.axis_index("c")
        pltpu.sync_copy(idx_hbm.at[wid], idx_v)
        copies = [
            pltpu.async_copy(
                table_hbm.at[idx_v.at[j]],
                rows_v.at[pl.ds(j * _CHUNK, _CHUNK)],
                sem,
            )
            for j in range(nch)
        ]
        for c in copies:
            c.wait()
        pltpu.sync_copy(rows_v, out_hbm.at[pl.ds(wid * bpw, bpw)])

    return gather_kernel(idx3, table)


def _tc_combine(coords2, rix2, w0t, b0r, g32):
    B = coords2.shape[0]
    blk = 2048
    grid = B // blk
    fs = jnp.asarray(_FREQS)
    sh = jnp.asarray(_SHIFTS)

    def body(c_ref, r_ref, w_ref, b_ref, f_ref, s_ref, g_ref, o_ref):
        c = c_ref[...]
        x = c * f_ref[...] + s_ref[...]
        e = jnp.sin(x)
        h = jnp.dot(e, w_ref[...], preferred_element_type=jnp.float32)
        h = jax.nn.sigmoid(h + b_ref[...])
        g = g_ref[...]
        o4 = r_ref[...] % 4
        w20 = jnp.where(
            o4 == 0,
            g[:, 0:_D_EMB],
            jnp.where(
                o4 == 1,
                g[:, 4 : 4 + _D_EMB],
                jnp.where(o4 == 2, g[:, 8 : 8 + _D_EMB], g[:, 12 : 12 + _D_EMB]),
            ),
        )
        o_ref[...] = jnp.sum(h * w20, axis=1, keepdims=True)

    return pl.pallas_call(
        body,
        grid=(grid,),
        in_specs=[
            pl.BlockSpec((blk, 1), lambda i: (i, 0)),
            pl.BlockSpec((blk, 1), lambda i: (i, 0)),
            pl.BlockSpec((_SINE_DIM, _D_EMB), lambda i: (0, 0)),
            pl.BlockSpec((1, _D_EMB), lambda i: (0, 0)),
            pl.BlockSpec((1, _SINE_DIM), lambda i: (0, 0)),
            pl.BlockSpec((1, _SINE_DIM), lambda i: (0, 0)),
            pl.BlockSpec((blk, 2 * _BLK_W), lambda i: (i, 0)),
        ],
        out_specs=pl.BlockSpec((blk, 1), lambda i: (i, 0)),
        out_shape=jax.ShapeDtypeStruct((B, 1), jnp.float32),
    )(coords2, rix2, w0t, b0r, fs, sh, g32)


def kernel(coordinates, region_ix, W0, b0, weight1):
    B = coordinates.shape[0]
    rix = region_ix.astype(jnp.int32)
    nwords = weight1.shape[0] * weight1.shape[1] * weight1.shape[2]
    table = weight1.reshape(nwords // _BLK_W, _BLK_W)
    b0_blk = (rix * _D_EMB) // _BLK_W
    blocks = jnp.stack([b0_blk, b0_blk + 1], axis=1).reshape(-1)
    g2 = _sc_gather(table, blocks)
    g32 = g2.reshape(B, 2 * _BLK_W)
    return _tc_combine(
        coordinates.reshape(B, 1),
        rix.reshape(B, 1),
        W0.T,
        b0.reshape(1, -1),
        g32,
    )
